# Initial kernel scaffold; baseline (speedup 1.0000x reference)
import jax
import jax.numpy as jnp
from jax.experimental import pallas as pl


def kernel(sets, E):
    raise NotImplementedError("write your pallas kernel here")


# speedup vs baseline: 1.7757x; 1.7757x over previous
import functools

import jax
import jax.numpy as jnp
from jax import lax
from jax.experimental import pallas as pl
from jax.experimental.pallas import tpu as pltpu
from jax.experimental.pallas import tpu_sc as plsc

BATCH = 16384
HIST = 50
DIM = 32

NC = 2
NS = 16
NW = NC * NS
B = BATCH * HIST
SUB = 128
NSUB = 10
CHUNK = SUB * NSUB
NCHUNKS = B // (NW * CHUNK)

_mesh = plsc.VectorSubcoreMesh(core_axis_name="c", subcore_axis_name="s")


@functools.partial(
    pl.kernel,
    mesh=_mesh,
    out_type=jax.ShapeDtypeStruct((NW * NCHUNKS, NSUB, SUB, DIM), jnp.float32),
    scratch_types=[
        pltpu.VMEM((NSUB, SUB), jnp.int32),
        pltpu.VMEM((NSUB, SUB, DIM), jnp.float32),
        pltpu.SemaphoreType.DMA,
    ],
    compiler_params=pltpu.CompilerParams(use_tc_tiling_on_sc=False),
)
def _sc_gather(idx_hbm, table_hbm, out_hbm, idx_v, rows_v, sem):
    wid = lax.axis_index("s") * NC + lax.axis_index("c")

    def body(c, carry):
        chunk_id = wid * NCHUNKS + c
        pltpu.sync_copy(idx_hbm.at[chunk_id], idx_v)
        copies = [
            pltpu.async_copy(table_hbm.at[idx_v.at[j]], rows_v.at[j], sem)
            for j in range(NSUB)
        ]
        for cp in copies:
            cp.wait()
        pltpu.sync_copy(rows_v, out_hbm.at[chunk_id])
        return carry

    lax.fori_loop(0, NCHUNKS, body, 0)


def kernel(sets, E):
    idx = sets.reshape(NW * NCHUNKS, NSUB, SUB)
    out = _sc_gather(idx, E)
    return out.reshape(BATCH, HIST, DIM)
